# baseline (speedup 1.0000x reference)
import functools

import jax
import jax.numpy as jnp
from jax import lax
from jax.experimental import pallas as pl
from jax.experimental.pallas import tpu as pltpu
from jax.experimental.pallas import tpu_sc as plsc

EPS = 1e-12
NC = 2
NS = 16
NW = NC * NS
L = 16
CHUNK = 128
PPT = 640

_GDN = lax.GatherDimensionNumbers(
    offset_dims=(), collapsed_slice_dims=(0,), start_index_map=(0,)
)


def _permute(v, p):
    return lax.gather(
        v, p[:, None], _GDN, slice_sizes=(1,),
        mode=lax.GatherScatterMode.PROMISE_IN_BOUNDS,
    )


def _xlane_sum(v, perms):
    for p in perms:
        v = v + _permute(v, p)
    return v


def _rsqrt(v):
    vi = lax.bitcast_convert_type(v, jnp.int32)
    yi = jnp.full((L,), 0x5F3759DF, jnp.int32) - lax.shift_right_arithmetic(vi, 1)
    y = lax.bitcast_convert_type(yi, jnp.float32)
    for _ in range(3):
        y = y * (1.5 - 0.5 * v * y * y)
    return y


def _sc_kernel(x_hbm, tt_hbm, sym_hbm, typ_hbm, g_hbm, b_hbm, out_hbm,
               xi, ti, ci, rows, combo, g_v, b_v,
               ix0, ix1, it0, it1, gs0, gs1, os0, os1,
               *, per_w, V, T, D):
    cid = lax.axis_index("c")
    sid = lax.axis_index("s")
    wid = sid * NC + cid
    w0 = wid * per_w
    pltpu.sync_copy(g_hbm, g_v)
    pltpu.sync_copy(b_hbm, b_v)
    nj = D // L
    gs = tuple(g_v[pl.ds(j * L, L)] for j in range(nj))
    bs = tuple(b_v[pl.ds(j * L, L)] for j in range(nj))
    lane = lax.iota(jnp.int32, L)
    perms = tuple(jnp.bitwise_xor(lane, k) for k in (8, 4, 2, 1))
    n = per_w // CHUNK
    ixsems = (ix0, ix1)
    itsems = (it0, it1)
    gsems = (gs0, gs1)
    osems = (os0, os1)

    p0 = sid * PPT
    for batch in range(PPT // CHUNK):
        pb = p0 + batch * CHUNK
        for g in range(CHUNK // L):
            pv = lane + (pb + g * L)
            q = pv // T
            xi[0, pl.ds(g * L, L)] = jnp.minimum(q, V - 1)
            ti[0, pl.ds(g * L, L)] = pv - q * T
        cps = pltpu.make_async_copy(sym_hbm.at[xi.at[0]], rows.at[0], gs0)
        cpt = pltpu.make_async_copy(typ_hbm.at[ti.at[0]], rows.at[1], gs1)
        cps.start()
        cpt.start()
        cps.wait()
        cpt.wait()
        r0, r1 = rows.at[0], rows.at[1]

        @plsc.parallel_loop(0, CHUNK, 1, unroll=4)
        def _pair(i):
            es = []
            for j in range(nj):
                es.append(r0[i, pl.ds(j * L, L)] + r1[i, pl.ds(j * L, L)])
            acc = es[0]
            for j in range(1, nj):
                acc = acc + es[j]
            acc2 = es[0] * es[0]
            for j in range(1, nj):
                acc2 = acc2 + es[j] * es[j]
            mean = _xlane_sum(acc, perms) * (1.0 / D)
            meansq = _xlane_sum(acc2, perms) * (1.0 / D)
            var = meansq - mean * mean
            rstd = _rsqrt(var + EPS)
            mrs = mean * rstd
            for j in range(nj):
                a = gs[j] * rstd
                cc = bs[j] - gs[j] * mrs
                r0[i, pl.ds(j * L, L)] = es[j] * a + cc

        st = pltpu.make_async_copy(rows.at[0], combo.at[pl.ds(pb, CHUNK)], os0)
        st.start()
        st.wait()
    plsc.subcore_barrier()

    def _idxcopies(c, b):
        src_x = x_hbm.at[pl.ds(w0 + c * CHUNK, CHUNK)]
        src_t = tt_hbm.at[pl.ds(w0 + c * CHUNK, CHUNK)]
        cpx = pltpu.make_async_copy(src_x, xi.at[b], ixsems[b])
        cpt = pltpu.make_async_copy(src_t, ti.at[b], itsems[b])
        return cpx, cpt

    def _gather(b):
        return pltpu.make_async_copy(combo.at[ci.at[b]], rows.at[b], gsems[b])

    def _outcopy(c, b):
        dst = out_hbm.at[pl.ds(w0 + c * CHUNK, CHUNK)]
        return pltpu.make_async_copy(rows.at[b], dst, osems[b])

    for b in range(2):
        cpx, cpt = _idxcopies(b, b)
        cpx.start()
        cpt.start()

    def pair_body(k, carry):
        for b in range(2):
            c = 2 * k + b
            cpx, cpt = _idxcopies(c, b)
            cpx.wait()
            cpt.wait()

            for g in range(CHUNK // L):
                xv = xi[b, pl.ds(g * L, L)]
                tv = ti[b, pl.ds(g * L, L)]
                ci[b, pl.ds(g * L, L)] = xv * T + tv

            @pl.when(c + 2 < n)
            def _():
                cpx2, cpt2 = _idxcopies(c + 2, b)
                cpx2.start()
                cpt2.start()

            @pl.when(c >= 2)
            def _():
                _outcopy(c - 2, b).wait()

            g2 = _gather(b)
            g2.start()
            g2.wait()
            _outcopy(c, b).start()
        return carry

    lax.fori_loop(0, n // 2, pair_body, 0)
    for b in range(2):
        _outcopy(n - 2 + b, b).wait()


def kernel(x, token_types, symbol_table, token_type_table, gamma, beta):
    B, S = x.shape
    V, D = symbol_table.shape
    T = token_type_table.shape[0]
    N = B * S
    assert N % (NW * 2 * CHUNK) == 0
    assert NS * PPT >= V * T and PPT % CHUNK == 0
    per_w = N // NW

    xf = x.reshape(N).astype(jnp.int32)
    tf = token_types.reshape(N).astype(jnp.int32)

    mesh = plsc.VectorSubcoreMesh(
        core_axis_name="c", subcore_axis_name="s", num_cores=NC, num_subcores=NS
    )
    run = pl.kernel(
        functools.partial(_sc_kernel, per_w=per_w, V=V, T=T, D=D),
        out_type=jax.ShapeDtypeStruct((N, D), jnp.float32),
        mesh=mesh,
        compiler_params=pltpu.CompilerParams(
            use_tc_tiling_on_sc=False, needs_layout_passes=False
        ),
        scratch_types=[
            pltpu.VMEM((2, CHUNK), jnp.int32),
            pltpu.VMEM((2, CHUNK), jnp.int32),
            pltpu.VMEM((2, CHUNK), jnp.int32),
            pltpu.VMEM((2, CHUNK, D), jnp.float32),
            pltpu.VMEM_SHARED((NS * PPT, D), jnp.float32),
            pltpu.VMEM((D,), jnp.float32),
            pltpu.VMEM((D,), jnp.float32),
        ] + [pltpu.SemaphoreType.DMA] * 8,
    )
    out = run(xf, tf, symbol_table, token_type_table, gamma, beta)
    return out.reshape(B, S, D)
